# baseline (speedup 1.0000x reference)
import functools

import jax
import jax.numpy as jnp
import numpy as np
from jax import lax
from jax.experimental import pallas as pl
from jax.experimental.pallas import tpu as pltpu
from jax.experimental.pallas import tpu_sc as plsc

B = 128
V = 100000
TOP_P = 0.9
R = 8
GRID = B // R

G = 128
NGR = 782
VPAD = NGR * G
NGR_PAD = 896
CAP = 64
GIDBUF = 96
CAPE = 256
EBUF = 272
NTILES = 32
ROWS_PER_TILE = B // NTILES

_U32 = np.uint32
_TINY = np.float32(1.1754943508222875e-38)
_K0 = _U32(0)
_K1 = _U32(42)
_K2 = _U32(0x1BD11BDA ^ 42)


def _rowsum(x):
    n = x.shape[1]
    ones = jnp.ones((n, 1), jnp.float32)
    return jax.lax.dot_general(x, ones, (((1,), (0,)), ((), ())),
                               preferred_element_type=jnp.float32)


def _monotone_key(x):
    u = lax.bitcast_convert_type(x, jnp.uint32)
    neg = (u >> _U32(31)) == _U32(1)
    return jnp.where(neg, ~u, u | _U32(0x80000000))


def _key_to_float(key):
    hi = key >= _U32(0x80000000)
    bits = jnp.where(hi, key ^ _U32(0x80000000), ~key)
    return lax.bitcast_convert_type(bits, jnp.float32)


def _rotl(x, r):
    return (x << _U32(r)) | (x >> _U32(32 - r))


def _threefry_rounds(x0, x1, rots):
    for r in rots:
        x0 = x0 + x1
        x1 = _rotl(x1, r) ^ x0
    return x0, x1


def _gumbel(flat_idx_u32):
    x0 = jnp.zeros_like(flat_idx_u32) + _K0
    x1 = flat_idx_u32 + _K1
    x0, x1 = _threefry_rounds(x0, x1, (13, 15, 26, 6))
    x0, x1 = _threefry_rounds(x0 + _K1, x1 + _K2 + _U32(1), (17, 29, 16, 24))
    x0, x1 = _threefry_rounds(x0 + _K2, x1 + _K0 + _U32(2), (13, 15, 26, 6))
    x0, x1 = _threefry_rounds(x0 + _K0, x1 + _K1 + _U32(3), (17, 29, 16, 24))
    x0, x1 = _threefry_rounds(x0 + _K1, x1 + _K2 + _U32(4), (13, 15, 26, 6))
    bits = (x0 + _K2) ^ (x1 + _K0 + _U32(5))
    fb = (bits >> _U32(9)) | _U32(0x3F800000)
    floats = lax.bitcast_convert_type(fb, jnp.float32) - jnp.float32(1.0)
    u = jnp.maximum(_TINY, floats + _TINY)
    return -jnp.log(-jnp.log(u))


def _k1_body(k_ref, x3_ref, gmax_ref, t0_ref, mx_ref):
    kf = jnp.float32(k_ref[0])
    x3 = x3_ref[...]
    g = jnp.max(x3, axis=2)
    mx_ref[...] = jnp.max(g, axis=1, keepdims=True)
    gmax_ref[:, :NGR] = g
    gmax_ref[:, NGR:] = jnp.full((R, NGR_PAD - NGR), -jnp.inf, jnp.float32)

    km = _monotone_key(g)

    def bs(_, lohi):
        lo, hi = lohi
        mid = lo + ((hi - lo) >> _U32(1))
        cnt = _rowsum((km >= mid).astype(jnp.float32))
        ge_k = cnt >= kf
        return jnp.where(ge_k, mid, lo), jnp.where(ge_k, hi, mid)

    lo0 = jnp.zeros((R, 1), jnp.uint32)
    hi0 = jnp.full((R, 1), _U32(0xFFFFFFFF))
    t0_key, _ = lax.fori_loop(0, 32, bs, (lo0, hi0))
    t0_ref[...] = jnp.broadcast_to(_key_to_float(t0_key), (R, 16))


def _k2_body(gmax_hbm, t0_hbm, rowbase_hbm, logits_hbm,
             evals_hbm, epos_hbm, gids_hbm, cnts_hbm,
             row_buf, t0_buf, base_buf, gid_buf, gidx_buf, rows_v, cnt_buf,
             off_buf, ids_buf, evals_buf, epos_buf, sem):
    wid = lax.axis_index("s") * 2 + lax.axis_index("c")
    zeros16 = jnp.zeros((16,), jnp.int32)
    zf16 = jnp.zeros((16,), jnp.float32)
    for j in range(ROWS_PER_TILE):
        r = wid * ROWS_PER_TILE + j
        pltpu.sync_copy(gmax_hbm.at[pl.ds(r * NGR_PAD, NGR_PAD)], row_buf)
        pltpu.sync_copy(t0_hbm.at[pl.ds(r * 16, 16)], t0_buf)
        pltpu.sync_copy(rowbase_hbm.at[pl.ds(r * 16, 16)], base_buf)
        for z in range(GIDBUF // 16):
            gid_buf[pl.ds(z * 16, 16)] = zeros16
        off_buf[...] = zeros16
        ids_buf[...] = lax.iota(jnp.int32, 16)

        def step(s, carry):
            m = row_buf[pl.ds(s * 16, 16)]
            msk = m >= t0_buf[...]
            off_v = off_buf[...]
            ids_v = ids_buf[...]
            cum = jnp.cumsum(msk.astype(jnp.int32))
            pos = jnp.minimum(off_v + cum - 1, GIDBUF - 1)
            plsc.store_scatter(gid_buf, [pos], ids_v, mask=msk)
            off_buf[...] = off_v + plsc.all_reduce_population_count(msk)
            ids_buf[...] = ids_v + 16
            return carry

        lax.fori_loop(0, NGR_PAD // 16, step, jnp.int32(0))
        cnt_buf[...] = jnp.minimum(off_buf[...], CAP)
        base_v = base_buf[...]
        for z in range(CAP // 16):
            gidx_buf[pl.ds(z * 16, 16)] = gid_buf[pl.ds(z * 16, 16)] + base_v
        pltpu.async_copy(logits_hbm.at[gidx_buf], rows_v, sem).wait()

        for z in range(EBUF // 16):
            evals_buf[pl.ds(z * 16, 16)] = zf16
            epos_buf[pl.ds(z * 16, 16)] = zeros16
        off_buf[...] = zeros16
        ids_buf[...] = lax.iota(jnp.int32, 16)
        cntv = cnt_buf[...]

        def estep(s, carry):
            fp_v = ids_buf[...]
            m = rows_v[s >> 3, pl.ds((s & 7) * 16, 16)]
            msk = (m >= t0_buf[...]) & ((fp_v >> 7) < cntv)
            off_v = off_buf[...]
            cum = jnp.cumsum(msk.astype(jnp.int32))
            pos = jnp.minimum(off_v + cum - 1, EBUF - 1)
            plsc.store_scatter(evals_buf, [pos], m, mask=msk)
            plsc.store_scatter(epos_buf, [pos], fp_v, mask=msk)
            off_buf[...] = off_v + plsc.all_reduce_population_count(msk)
            ids_buf[...] = fp_v + 16
            return carry

        lax.fori_loop(0, CAP * G // 16, estep, jnp.int32(0))
        cnt_buf[...] = jnp.minimum(off_buf[...], CAPE)
        pltpu.sync_copy(evals_buf.at[pl.ds(0, CAPE)],
                        evals_hbm.at[pl.ds(r * CAPE, CAPE)])
        pltpu.sync_copy(epos_buf.at[pl.ds(0, CAPE)],
                        epos_hbm.at[pl.ds(r * CAPE, CAPE)])
        pltpu.sync_copy(gid_buf.at[pl.ds(0, CAP)],
                        gids_hbm.at[pl.ds(r * CAP, CAP)])
        pltpu.sync_copy(cnt_buf, cnts_hbm.at[pl.ds(r * 16, 16)])


def _k3_body(k_ref, x_ref, ev_ref, ep_ref, gid_ref, cnt_ref, mx_ref, tok_ref,
             probs_ref):
    i = pl.program_id(0)
    k = k_ref[0]

    cx = ev_ref[...]
    fp = ep_ref[...]
    gids = gid_ref[...]
    cnt = cnt_ref[:, 0:1]
    mx = mx_ref[...]

    slot = fp >> 7
    onehot = (slot[:, :, None] == lax.broadcasted_iota(
        jnp.int32, (R, CAPE, CAP), 2)).astype(jnp.int32)
    colseg = jnp.sum(onehot * gids[:, None, :], axis=2)
    col = colseg * G + (fp & (G - 1))

    valid = lax.broadcasted_iota(jnp.int32, (R, CAPE), 1) < cnt
    ckm = jnp.where(valid, _monotone_key(cx), _U32(0))

    kf = jnp.float32(k)

    def bs1(_, lohi):
        lo, hi = lohi
        mid = lo + ((hi - lo) >> _U32(1))
        cn = _rowsum((ckm >= mid).astype(jnp.float32))
        ge_k = cn >= kf
        return jnp.where(ge_k, mid, lo), jnp.where(ge_k, hi, mid)

    lo0 = jnp.zeros((R, 1), jnp.uint32)
    hi0 = jnp.full((R, 1), _U32(0xFFFFFFFF))
    kth_key, _ = lax.fori_loop(0, 32, bs1, (lo0, hi0))

    e = jnp.where(ckm >= kth_key, jnp.exp(cx - mx), jnp.float32(0.0))
    s_total = jnp.sum(e, axis=1, keepdims=True)
    q = e / s_total

    def bs2(_, lohi):
        lo, hi = lohi
        mid = lo + ((hi - lo) >> _U32(1))
        mass_gt = jnp.sum(jnp.where(ckm > mid, q, 0.0), axis=1, keepdims=True)
        keep = mass_gt <= jnp.float32(TOP_P)
        return jnp.where(keep, lo, mid), jnp.where(keep, mid, hi)

    _, cut_key = lax.fori_loop(0, 32, bs2, (lo0, hi0))

    strict = ckm > cut_key
    tie = ckm == cut_key
    mass_gt = jnp.sum(jnp.where(strict, q, 0.0), axis=1, keepdims=True)
    e_tie = jnp.max(jnp.where(tie, e, 0.0), axis=1, keepdims=True)
    q_tie = e_tie / s_total
    tie_cnt = _rowsum(tie.astype(jnp.float32)).astype(jnp.int32)

    def tie_loop(_, carry):
        c, rk = carry
        take = (c <= jnp.float32(TOP_P)) & (rk < tie_cnt)
        return c + q_tie, rk + take.astype(jnp.int32)

    _, r_keep = lax.fori_loop(
        0, 64, tie_loop, (mass_gt, jnp.zeros((R, 1), jnp.int32)))

    rkf = r_keep.astype(jnp.float32)

    def bs3(_, lohi):
        lo, hi = lohi
        mid = lo + ((hi - lo) >> 1)
        cn = _rowsum((tie & (col <= mid)).astype(jnp.float32))
        ok = cn >= rkf
        return jnp.where(ok, lo, mid), jnp.where(ok, mid, hi)

    lo3 = jnp.full((R, 1), jnp.int32(-1))
    hi3 = jnp.full((R, 1), jnp.int32(V - 1))
    _, m_cut = lax.fori_loop(0, 18, bs3, (lo3, hi3))

    kept_c = strict | (tie & (col <= m_cut))
    denom = jnp.sum(jnp.where(kept_c, e, 0.0), axis=1, keepdims=True)

    row2 = lax.broadcasted_iota(jnp.int32, (R, CAPE), 0)
    flat = ((i * R + row2) * V + col).astype(jnp.uint32)
    g = _gumbel(flat)
    score = jnp.where(kept_c, cx + g, jnp.float32(-jnp.inf))
    smax = jnp.max(score, axis=1, keepdims=True)
    tok = jnp.min(jnp.where(score == smax, col, jnp.int32(V)), axis=1,
                  keepdims=True)
    tok_ref[...] = tok

    x = x_ref[...]
    km = _monotone_key(x)
    colf = lax.broadcasted_iota(jnp.int32, (R, V), 1)
    kept = (km > cut_key) | ((km == cut_key) & (colf <= m_cut))
    probs_ref[...] = jnp.where(kept, jnp.exp(x - mx) / denom,
                               jnp.float32(0.0))


@jax.jit
def kernel(logits, top_k):
    kvec = jnp.reshape(top_k, (1,)).astype(jnp.int32)

    lpad = jnp.pad(logits, ((0, 0), (0, VPAD - V)),
                   constant_values=-jnp.inf)
    lpad3 = lpad.reshape(B, NGR, G)
    lpadN = lpad.reshape(B * NGR, G)
    rowbase = jnp.broadcast_to((jnp.arange(B, dtype=jnp.int32) * NGR)[:, None],
                               (B, 16))

    gmax, t0, mx = pl.pallas_call(
        _k1_body,
        grid_spec=pltpu.PrefetchScalarGridSpec(
            num_scalar_prefetch=1,
            grid=(GRID,),
            in_specs=[pl.BlockSpec((R, NGR, G), lambda i, kref: (i, 0, 0))],
            out_specs=[
                pl.BlockSpec((R, NGR_PAD), lambda i, kref: (i, 0)),
                pl.BlockSpec((R, 16), lambda i, kref: (i, 0)),
                pl.BlockSpec((R, 1), lambda i, kref: (i, 0)),
            ],
        ),
        out_shape=[
            jax.ShapeDtypeStruct((B, NGR_PAD), jnp.float32),
            jax.ShapeDtypeStruct((B, 16), jnp.float32),
            jax.ShapeDtypeStruct((B, 1), jnp.float32),
        ],
    )(kvec, lpad3)

    sc_mesh = plsc.VectorSubcoreMesh(core_axis_name="c", subcore_axis_name="s",
                                     num_cores=2, num_subcores=16)
    evals1, epos1, gids1, cnts1 = pl.kernel(
        _k2_body,
        out_type=[
            jax.ShapeDtypeStruct((B * CAPE,), jnp.float32),
            jax.ShapeDtypeStruct((B * CAPE,), jnp.int32),
            jax.ShapeDtypeStruct((B * CAP,), jnp.int32),
            jax.ShapeDtypeStruct((B * 16,), jnp.int32),
        ],
        mesh=sc_mesh,
        compiler_params=pltpu.CompilerParams(needs_layout_passes=False),
        scratch_types=[
            pltpu.VMEM((NGR_PAD,), jnp.float32),
            pltpu.VMEM((16,), jnp.float32),
            pltpu.VMEM((16,), jnp.int32),
            pltpu.VMEM((GIDBUF,), jnp.int32),
            pltpu.VMEM((CAP,), jnp.int32),
            pltpu.VMEM((CAP, G), jnp.float32),
            pltpu.VMEM((16,), jnp.int32),
            pltpu.VMEM((16,), jnp.int32),
            pltpu.VMEM((16,), jnp.int32),
            pltpu.VMEM((EBUF,), jnp.float32),
            pltpu.VMEM((EBUF,), jnp.int32),
            pltpu.SemaphoreType.DMA,
        ],
    )(gmax.reshape(B * NGR_PAD), t0.reshape(B * 16),
      rowbase.reshape(B * 16), lpadN)
    evals = evals1.reshape(B, CAPE)
    epos = epos1.reshape(B, CAPE)
    gids = gids1.reshape(B, CAP)
    cnts = cnts1.reshape(B, 16)

    tok2d, probs = pl.pallas_call(
        _k3_body,
        grid_spec=pltpu.PrefetchScalarGridSpec(
            num_scalar_prefetch=1,
            grid=(GRID,),
            in_specs=[
                pl.BlockSpec((R, V), lambda i, kref: (i, 0)),
                pl.BlockSpec((R, CAPE), lambda i, kref: (i, 0)),
                pl.BlockSpec((R, CAPE), lambda i, kref: (i, 0)),
                pl.BlockSpec((R, CAP), lambda i, kref: (i, 0)),
                pl.BlockSpec((R, 16), lambda i, kref: (i, 0)),
                pl.BlockSpec((R, 1), lambda i, kref: (i, 0)),
            ],
            out_specs=[
                pl.BlockSpec((R, 1), lambda i, kref: (i, 0)),
                pl.BlockSpec((R, V), lambda i, kref: (i, 0)),
            ],
        ),
        out_shape=[
            jax.ShapeDtypeStruct((B, 1), jnp.int32),
            jax.ShapeDtypeStruct((B, V), jnp.float32),
        ],
    )(kvec, logits, evals, epos, gids, cnts, mx)
    return tok2d[:, 0], probs


# speedup vs baseline: 1.2415x; 1.2415x over previous
import functools

import jax
import jax.numpy as jnp
import numpy as np
from jax import lax
from jax.experimental import pallas as pl
from jax.experimental.pallas import tpu as pltpu
from jax.experimental.pallas import tpu_sc as plsc

B = 128
V = 100000
TOP_P = 0.9
R = 8
GRID = B // R

G = 128
NGR = 782
VPAD = NGR * G
NGR_PAD = 896
CAP = 64
GIDBUF = 96
CAPE = 256
EBUF = 272
NTILES = 32
ROWS_PER_TILE = B // NTILES

_U32 = np.uint32
_TINY = np.float32(1.1754943508222875e-38)
_K0 = _U32(0)
_K1 = _U32(42)
_K2 = _U32(0x1BD11BDA ^ 42)


def _rowsum(x):
    n = x.shape[1]
    ones = jnp.ones((n, 1), jnp.float32)
    return jax.lax.dot_general(x, ones, (((1,), (0,)), ((), ())),
                               preferred_element_type=jnp.float32)


def _monotone_key(x):
    u = lax.bitcast_convert_type(x, jnp.uint32)
    neg = (u >> _U32(31)) == _U32(1)
    return jnp.where(neg, ~u, u | _U32(0x80000000))


def _key_to_float(key):
    hi = key >= _U32(0x80000000)
    bits = jnp.where(hi, key ^ _U32(0x80000000), ~key)
    return lax.bitcast_convert_type(bits, jnp.float32)


def _rotl(x, r):
    return (x << _U32(r)) | (x >> _U32(32 - r))


def _threefry_rounds(x0, x1, rots):
    for r in rots:
        x0 = x0 + x1
        x1 = _rotl(x1, r) ^ x0
    return x0, x1


def _gumbel(flat_idx_u32):
    x0 = jnp.zeros_like(flat_idx_u32) + _K0
    x1 = flat_idx_u32 + _K1
    x0, x1 = _threefry_rounds(x0, x1, (13, 15, 26, 6))
    x0, x1 = _threefry_rounds(x0 + _K1, x1 + _K2 + _U32(1), (17, 29, 16, 24))
    x0, x1 = _threefry_rounds(x0 + _K2, x1 + _K0 + _U32(2), (13, 15, 26, 6))
    x0, x1 = _threefry_rounds(x0 + _K0, x1 + _K1 + _U32(3), (17, 29, 16, 24))
    x0, x1 = _threefry_rounds(x0 + _K1, x1 + _K2 + _U32(4), (13, 15, 26, 6))
    bits = (x0 + _K2) ^ (x1 + _K0 + _U32(5))
    fb = (bits >> _U32(9)) | _U32(0x3F800000)
    floats = lax.bitcast_convert_type(fb, jnp.float32) - jnp.float32(1.0)
    u = jnp.maximum(_TINY, floats + _TINY)
    return -jnp.log(-jnp.log(u))


def _k1_body(k_ref, x3_ref, gmax_ref, t0_ref, mx_ref):
    kf = jnp.float32(k_ref[0])
    x3 = x3_ref[...]
    g = jnp.max(x3, axis=2)
    mx_ref[...] = jnp.max(g, axis=1, keepdims=True)
    gmax_ref[:, :NGR] = g
    gmax_ref[:, NGR:] = jnp.full((R, NGR_PAD - NGR), -jnp.inf, jnp.float32)

    km = _monotone_key(g)

    def bs(_, lohi):
        lo, hi = lohi
        mid = lo + ((hi - lo) >> _U32(1))
        cnt = _rowsum((km >= mid).astype(jnp.float32))
        ge_k = cnt >= kf
        return jnp.where(ge_k, mid, lo), jnp.where(ge_k, hi, mid)

    lo0 = jnp.zeros((R, 1), jnp.uint32)
    hi0 = jnp.full((R, 1), _U32(0xFFFFFFFF))
    t0_key, _ = lax.fori_loop(0, 16, bs, (lo0, hi0))
    t0_ref[...] = jnp.broadcast_to(_key_to_float(t0_key), (R, 16))


def _k2_body(gmax_hbm, t0_hbm, rowbase_hbm, logits_hbm,
             evals_hbm, epos_hbm, gids_hbm, cnts_hbm,
             row_buf, t0_buf, base_buf, gid_buf, gidx_buf, rows_v, cnt_buf,
             off_buf, ids_buf, evals_buf, epos_buf, sem):
    wid = lax.axis_index("s") * 2 + lax.axis_index("c")
    zeros16 = jnp.zeros((16,), jnp.int32)
    zf16 = jnp.zeros((16,), jnp.float32)
    for j in range(ROWS_PER_TILE):
        r = wid * ROWS_PER_TILE + j
        pltpu.sync_copy(gmax_hbm.at[pl.ds(r * NGR_PAD, NGR_PAD)], row_buf)
        pltpu.sync_copy(t0_hbm.at[pl.ds(r * 16, 16)], t0_buf)
        pltpu.sync_copy(rowbase_hbm.at[pl.ds(r * 16, 16)], base_buf)
        for z in range(GIDBUF // 16):
            gid_buf[pl.ds(z * 16, 16)] = zeros16
        off_buf[...] = zeros16
        ids_buf[...] = lax.iota(jnp.int32, 16)

        def step(s, carry):
            m = row_buf[pl.ds(s * 16, 16)]
            msk = m >= t0_buf[...]
            off_v = off_buf[...]
            ids_v = ids_buf[...]
            cum = jnp.cumsum(msk.astype(jnp.int32))
            pos = jnp.minimum(off_v + cum - 1, GIDBUF - 1)
            plsc.store_scatter(gid_buf, [pos], ids_v, mask=msk)
            off_buf[...] = off_v + plsc.all_reduce_population_count(msk)
            ids_buf[...] = ids_v + 16
            return carry

        lax.fori_loop(0, NGR_PAD // 16, step, jnp.int32(0))
        cnt_buf[...] = jnp.minimum(off_buf[...], CAP)
        base_v = base_buf[...]
        for z in range(CAP // 16):
            gidx_buf[pl.ds(z * 16, 16)] = gid_buf[pl.ds(z * 16, 16)] + base_v
        pltpu.async_copy(logits_hbm.at[gidx_buf], rows_v, sem).wait()

        for z in range(EBUF // 16):
            evals_buf[pl.ds(z * 16, 16)] = zf16
            epos_buf[pl.ds(z * 16, 16)] = zeros16
        off_buf[...] = zeros16
        ids_buf[...] = lax.iota(jnp.int32, 16)
        cntv = cnt_buf[...]

        def estep(s, carry):
            fp_v = ids_buf[...]
            m = rows_v[s >> 3, pl.ds((s & 7) * 16, 16)]
            msk = (m >= t0_buf[...]) & ((fp_v >> 7) < cntv)
            off_v = off_buf[...]
            cum = jnp.cumsum(msk.astype(jnp.int32))
            pos = jnp.minimum(off_v + cum - 1, EBUF - 1)
            plsc.store_scatter(evals_buf, [pos], m, mask=msk)
            plsc.store_scatter(epos_buf, [pos], fp_v, mask=msk)
            off_buf[...] = off_v + plsc.all_reduce_population_count(msk)
            ids_buf[...] = fp_v + 16
            return carry

        lax.fori_loop(0, CAP * G // 16, estep, jnp.int32(0))
        cnt_buf[...] = jnp.minimum(off_buf[...], CAPE)
        pltpu.sync_copy(evals_buf.at[pl.ds(0, CAPE)],
                        evals_hbm.at[pl.ds(r * CAPE, CAPE)])
        pltpu.sync_copy(epos_buf.at[pl.ds(0, CAPE)],
                        epos_hbm.at[pl.ds(r * CAPE, CAPE)])
        pltpu.sync_copy(gid_buf.at[pl.ds(0, CAP)],
                        gids_hbm.at[pl.ds(r * CAP, CAP)])
        pltpu.sync_copy(cnt_buf, cnts_hbm.at[pl.ds(r * 16, 16)])


def _k3_body(k_ref, x_ref, ev_ref, ep_ref, gid_ref, cnt_ref, mx_ref, tok_ref,
             probs_ref):
    i = pl.program_id(0)
    k = k_ref[0]

    cx = ev_ref[...]
    fp = ep_ref[...]
    gids = gid_ref[...]
    cnt = cnt_ref[:, 0:1]
    mx = mx_ref[...]

    slot = fp >> 7
    onehot = (slot[:, :, None] == lax.broadcasted_iota(
        jnp.int32, (R, CAPE, CAP), 2)).astype(jnp.int32)
    colseg = jnp.sum(onehot * gids[:, None, :], axis=2)
    col = colseg * G + (fp & (G - 1))

    valid = lax.broadcasted_iota(jnp.int32, (R, CAPE), 1) < cnt
    ckm = jnp.where(valid, _monotone_key(cx), _U32(0))

    kf = jnp.float32(k)

    def bs1(_, lohi):
        lo, hi = lohi
        mid = lo + ((hi - lo) >> _U32(1))
        cn = _rowsum((ckm >= mid).astype(jnp.float32))
        ge_k = cn >= kf
        return jnp.where(ge_k, mid, lo), jnp.where(ge_k, hi, mid)

    lo0 = jnp.zeros((R, 1), jnp.uint32)
    hi0 = jnp.full((R, 1), _U32(0xFFFFFFFF))
    kth_key, _ = lax.fori_loop(0, 32, bs1, (lo0, hi0))

    e = jnp.where(ckm >= kth_key, jnp.exp(cx - mx), jnp.float32(0.0))
    s_total = jnp.sum(e, axis=1, keepdims=True)
    q = e / s_total

    def bs2(_, lohi):
        lo, hi = lohi
        mid = lo + ((hi - lo) >> _U32(1))
        mass_gt = jnp.sum(jnp.where(ckm > mid, q, 0.0), axis=1, keepdims=True)
        keep = mass_gt <= jnp.float32(TOP_P)
        return jnp.where(keep, lo, mid), jnp.where(keep, mid, hi)

    _, cut_key = lax.fori_loop(0, 32, bs2, (lo0, hi0))

    strict = ckm > cut_key
    tie = ckm == cut_key
    mass_gt = jnp.sum(jnp.where(strict, q, 0.0), axis=1, keepdims=True)
    e_tie = jnp.max(jnp.where(tie, e, 0.0), axis=1, keepdims=True)
    q_tie = e_tie / s_total
    tie_cnt = _rowsum(tie.astype(jnp.float32)).astype(jnp.int32)

    def tie_loop(_, carry):
        c, rk = carry
        take = (c <= jnp.float32(TOP_P)) & (rk < tie_cnt)
        return c + q_tie, rk + take.astype(jnp.int32)

    _, r_keep = lax.fori_loop(
        0, 64, tie_loop, (mass_gt, jnp.zeros((R, 1), jnp.int32)))

    rkf = r_keep.astype(jnp.float32)

    def bs3(_, lohi):
        lo, hi = lohi
        mid = lo + ((hi - lo) >> 1)
        cn = _rowsum((tie & (col <= mid)).astype(jnp.float32))
        ok = cn >= rkf
        return jnp.where(ok, lo, mid), jnp.where(ok, mid, hi)

    lo3 = jnp.full((R, 1), jnp.int32(-1))
    hi3 = jnp.full((R, 1), jnp.int32(V - 1))
    _, m_cut = lax.fori_loop(0, 18, bs3, (lo3, hi3))

    kept_c = strict | (tie & (col <= m_cut))
    denom = jnp.sum(jnp.where(kept_c, e, 0.0), axis=1, keepdims=True)

    row2 = lax.broadcasted_iota(jnp.int32, (R, CAPE), 0)
    flat = ((i * R + row2) * V + col).astype(jnp.uint32)
    g = _gumbel(flat)
    score = jnp.where(kept_c, cx + g, jnp.float32(-jnp.inf))
    smax = jnp.max(score, axis=1, keepdims=True)
    tok = jnp.min(jnp.where(score == smax, col, jnp.int32(V)), axis=1,
                  keepdims=True)
    tok_ref[...] = tok

    x = x_ref[...]
    km = _monotone_key(x)
    colf = lax.broadcasted_iota(jnp.int32, (R, V), 1)
    kept = (km > cut_key) | ((km == cut_key) & (colf <= m_cut))
    probs_ref[...] = jnp.where(kept, jnp.exp(x - mx) / denom,
                               jnp.float32(0.0))


@jax.jit
def kernel(logits, top_k):
    kvec = jnp.reshape(top_k, (1,)).astype(jnp.int32)

    lpad = jnp.pad(logits, ((0, 0), (0, VPAD - V)),
                   constant_values=-jnp.inf)
    lpad3 = lpad.reshape(B, NGR, G)
    lpadN = lpad.reshape(B * NGR, G)
    rowbase = jnp.broadcast_to((jnp.arange(B, dtype=jnp.int32) * NGR)[:, None],
                               (B, 16))

    gmax, t0, mx = pl.pallas_call(
        _k1_body,
        grid_spec=pltpu.PrefetchScalarGridSpec(
            num_scalar_prefetch=1,
            grid=(GRID,),
            in_specs=[pl.BlockSpec((R, NGR, G), lambda i, kref: (i, 0, 0))],
            out_specs=[
                pl.BlockSpec((R, NGR_PAD), lambda i, kref: (i, 0)),
                pl.BlockSpec((R, 16), lambda i, kref: (i, 0)),
                pl.BlockSpec((R, 1), lambda i, kref: (i, 0)),
            ],
        ),
        out_shape=[
            jax.ShapeDtypeStruct((B, NGR_PAD), jnp.float32),
            jax.ShapeDtypeStruct((B, 16), jnp.float32),
            jax.ShapeDtypeStruct((B, 1), jnp.float32),
        ],
    )(kvec, lpad3)

    sc_mesh = plsc.VectorSubcoreMesh(core_axis_name="c", subcore_axis_name="s",
                                     num_cores=2, num_subcores=16)
    evals1, epos1, gids1, cnts1 = pl.kernel(
        _k2_body,
        out_type=[
            jax.ShapeDtypeStruct((B * CAPE,), jnp.float32),
            jax.ShapeDtypeStruct((B * CAPE,), jnp.int32),
            jax.ShapeDtypeStruct((B * CAP,), jnp.int32),
            jax.ShapeDtypeStruct((B * 16,), jnp.int32),
        ],
        mesh=sc_mesh,
        compiler_params=pltpu.CompilerParams(needs_layout_passes=False),
        scratch_types=[
            pltpu.VMEM((NGR_PAD,), jnp.float32),
            pltpu.VMEM((16,), jnp.float32),
            pltpu.VMEM((16,), jnp.int32),
            pltpu.VMEM((GIDBUF,), jnp.int32),
            pltpu.VMEM((CAP,), jnp.int32),
            pltpu.VMEM((CAP, G), jnp.float32),
            pltpu.VMEM((16,), jnp.int32),
            pltpu.VMEM((16,), jnp.int32),
            pltpu.VMEM((16,), jnp.int32),
            pltpu.VMEM((EBUF,), jnp.float32),
            pltpu.VMEM((EBUF,), jnp.int32),
            pltpu.SemaphoreType.DMA,
        ],
    )(gmax.reshape(B * NGR_PAD), t0.reshape(B * 16),
      rowbase.reshape(B * 16), lpadN)
    evals = evals1.reshape(B, CAPE)
    epos = epos1.reshape(B, CAPE)
    gids = gids1.reshape(B, CAP)
    cnts = cnts1.reshape(B, 16)

    tok2d, probs = pl.pallas_call(
        _k3_body,
        grid_spec=pltpu.PrefetchScalarGridSpec(
            num_scalar_prefetch=1,
            grid=(GRID,),
            in_specs=[
                pl.BlockSpec((R, V), lambda i, kref: (i, 0)),
                pl.BlockSpec((R, CAPE), lambda i, kref: (i, 0)),
                pl.BlockSpec((R, CAPE), lambda i, kref: (i, 0)),
                pl.BlockSpec((R, CAP), lambda i, kref: (i, 0)),
                pl.BlockSpec((R, 16), lambda i, kref: (i, 0)),
                pl.BlockSpec((R, 1), lambda i, kref: (i, 0)),
            ],
            out_specs=[
                pl.BlockSpec((R, 1), lambda i, kref: (i, 0)),
                pl.BlockSpec((R, V), lambda i, kref: (i, 0)),
            ],
        ),
        out_shape=[
            jax.ShapeDtypeStruct((B, 1), jnp.int32),
            jax.ShapeDtypeStruct((B, V), jnp.float32),
        ],
    )(kvec, logits, evals, epos, gids, cnts, mx)
    return tok2d[:, 0], probs
